# baseline (speedup 1.0000x reference)
import functools

import numpy as np
import jax
import jax.numpy as jnp
from jax import lax
from jax.experimental import pallas as pl
from jax.experimental.pallas import tpu as pltpu

_BN_EPS = 1e-5
_SUB = 128
_JB = 8
_VMEM_LIMIT = 48 * 1024 * 1024


@functools.lru_cache(None)
def _tap_selector(k, stride, pad, h, w):
    ho = (h + 2 * pad - k) // stride + 1
    wo = (w + 2 * pad - k) // stride + 1
    t = np.zeros((k * k, ho * wo, h * w), np.float32)
    for kh in range(k):
        for kw in range(k):
            for oy in range(ho):
                for ox in range(wo):
                    iy = oy * stride + kh - pad
                    ix = ox * stride + kw - pad
                    if 0 <= iy < h and 0 <= ix < w:
                        t[kh * k + kw, oy * wo + ox, iy * w + ix] = 1.0
    return t, ho, wo


@functools.lru_cache(None)
def _sel1_flat(k, stride, pad, h, w):
    t, ho, wo = _tap_selector(k, stride, pad, h, w)
    return t.reshape(k * k, -1), ho, wo


@functools.lru_cache(None)
def _sel2_flat(k, stride, pad, h, w, cin):
    t, ho, wo = _tap_selector(k, stride, pad, h, w)
    p, hw = ho * wo, h * w
    s = np.zeros((cin, k * k, p, cin, hw), np.float32)
    for d in range(cin):
        s[d, :, :, d, :] = t
    return s.reshape(cin * k * k, p * cin * hw), ho, wo


def _round_up(a, m):
    return ((a + m - 1) // m) * m


def _p1_body(x_ref, m1_ref, s_ref, q_ref, g_acc, sx_acc):
    i = pl.program_id(0)

    @pl.when(i == 0)
    def _init():
        g_acc[...] = jnp.zeros_like(g_acc)
        sx_acc[...] = jnp.zeros_like(sx_acc)

    for j in range(x_ref.shape[1]):
        xj = x_ref[:, j, :]
        g_acc[...] += lax.dot_general(xj, xj, (((1,), (1,)), ((), ())),
                                      preferred_element_type=jnp.float32)
        sx_acc[...] += jnp.sum(xj, axis=1, keepdims=True)

    @pl.when(i == pl.num_programs(0) - 1)
    def _fin():
        m1 = m1_ref[...]
        e = jnp.dot(m1, g_acc[...], preferred_element_type=jnp.float32)
        s_ref[...] = jnp.dot(m1, sx_acc[...], preferred_element_type=jnp.float32)
        q_ref[...] = jnp.sum(e * m1, axis=1, keepdims=True)


def _p2_body(x_ref, m1_ref, s1_ref, t1_ref, m2_ref, y_ref, s_ref, q_ref):
    s = jnp.zeros((y_ref.shape[0], 1), jnp.float32)
    q = jnp.zeros((y_ref.shape[0], 1), jnp.float32)
    for j in range(x_ref.shape[1]):
        z1 = jnp.dot(m1_ref[...], x_ref[:, j, :],
                     preferred_element_type=jnp.float32)
        a1 = jnp.maximum(z1 * s1_ref[...] + t1_ref[...], 0.0)
        z2 = jnp.dot(m2_ref[...], a1, preferred_element_type=jnp.float32)
        y_ref[:, j, :] = z2.astype(y_ref.dtype)
        s += jnp.sum(z2, axis=1, keepdims=True)
        q += jnp.sum(z2 * z2, axis=1, keepdims=True)
    s_ref[...] = s[None]
    q_ref[...] = q[None]


def _p3_body(y_ref, s2_ref, t2_ref, w1_ref, b1_ref, w2_ref, b2_ref,
             lg_ref, lp_ref):
    lgs = []
    lps = []
    for j in range(y_ref.shape[1]):
        a2 = jnp.maximum(y_ref[:, j, :] * s2_ref[...] + t2_ref[...], 0.0)
        h = jnp.dot(w1_ref[...], a2,
                    preferred_element_type=jnp.float32) + b1_ref[...]
        lg = jnp.dot(w2_ref[...], h,
                     preferred_element_type=jnp.float32) + b2_ref[...]
        m = jnp.max(lg, axis=0, keepdims=True)
        sh = lg - m
        lp = sh - jnp.log(jnp.sum(jnp.exp(sh), axis=0, keepdims=True))
        lgs.append(lg.T)
        lps.append(lp.T)
    lg_ref[...] = jnp.concatenate(lgs, axis=0)
    lp_ref[...] = jnp.concatenate(lps, axis=0)


def _bn_cols(s, q, count, gamma, beta, channels, positions, corr=None):
    if corr is not None:
        s = s - corr[0]
        q = q - corr[1]
    sc = s.reshape(channels, positions).sum(axis=1)
    qc = q.reshape(channels, positions).sum(axis=1)
    mean = sc / count
    var = qc / count - mean * mean
    scale = gamma * lax.rsqrt(var + _BN_EPS)
    shift = beta - mean * scale
    return (jnp.repeat(scale, positions).reshape(-1, 1),
            jnp.repeat(shift, positions).reshape(-1, 1))


def kernel(x, conv1_w, conv1_b, bn1_g, bn1_b, conv2_w, conv2_b, bn2_g, bn2_b,
           fc1_w, fc1_b, fc2_w, fc2_b):
    n, cin, h, w = x.shape
    sel1, h1, w1 = _sel1_flat(3, 2, 1, h, w)
    c1 = conv1_w.shape[0]
    c2 = conv2_w.shape[0]
    sel2, h2, w2 = _sel2_flat(3, 2, 1, h1, w1, c1)
    p1 = h1 * w1
    p2 = h2 * w2
    f0 = cin * h * w
    f1 = c1 * p1
    f2 = c2 * p2
    o = fc2_w.shape[0]
    hdim = fc1_w.shape[0]
    tile = _JB * _SUB

    m1 = jnp.dot(conv1_w.reshape(c1, cin * 9), jnp.asarray(sel1)).reshape(f1, f0)
    m2 = jnp.dot(conv2_w.reshape(c2, c1 * 9), jnp.asarray(sel2)).reshape(f2, f1)

    npad = _round_up(n, tile)
    if npad == n:
        xv = jnp.transpose(x, (2, 3, 1, 0)).reshape(f0, n // _SUB, _SUB)
    else:
        xf = jnp.transpose(x, (2, 3, 1, 0)).reshape(f0, n)
        xv = jnp.pad(xf, ((0, 0), (0, npad - n))).reshape(f0, npad // _SUB, _SUB)
    g = npad // tile

    params = pltpu.CompilerParams(
        dimension_semantics=("parallel",), vmem_limit_bytes=_VMEM_LIMIT)
    params_seq = pltpu.CompilerParams(
        dimension_semantics=("arbitrary",), vmem_limit_bytes=_VMEM_LIMIT)

    s1f, q1f = pl.pallas_call(
        _p1_body,
        out_shape=(jax.ShapeDtypeStruct((f1, 1), jnp.float32),
                   jax.ShapeDtypeStruct((f1, 1), jnp.float32)),
        grid=(g,),
        in_specs=[pl.BlockSpec((f0, _JB, _SUB), lambda i: (0, i, 0)),
                  pl.BlockSpec((f1, f0), lambda i: (0, 0))],
        out_specs=(pl.BlockSpec((f1, 1), lambda i: (0, 0)),
                   pl.BlockSpec((f1, 1), lambda i: (0, 0))),
        scratch_shapes=[pltpu.VMEM((f0, f0), jnp.float32),
                        pltpu.VMEM((f0, 1), jnp.float32)],
        compiler_params=params_seq,
    )(xv, m1)
    s1col, t1col = _bn_cols(s1f[:, 0], q1f[:, 0], float(n * p1),
                            bn1_g, bn1_b, c1, p1)

    y2, s2p, q2p = pl.pallas_call(
        _p2_body,
        out_shape=(jax.ShapeDtypeStruct((f2, npad // _SUB, _SUB), jnp.bfloat16),
                   jax.ShapeDtypeStruct((g, f2, 1), jnp.float32),
                   jax.ShapeDtypeStruct((g, f2, 1), jnp.float32)),
        grid=(g,),
        in_specs=[pl.BlockSpec((f0, _JB, _SUB), lambda i: (0, i, 0)),
                  pl.BlockSpec((f1, f0), lambda i: (0, 0)),
                  pl.BlockSpec((f1, 1), lambda i: (0, 0)),
                  pl.BlockSpec((f1, 1), lambda i: (0, 0)),
                  pl.BlockSpec((f2, f1), lambda i: (0, 0))],
        out_specs=(pl.BlockSpec((f2, _JB, _SUB), lambda i: (0, i, 0)),
                   pl.BlockSpec((1, f2, 1), lambda i: (i, 0, 0)),
                   pl.BlockSpec((1, f2, 1), lambda i: (i, 0, 0))),
        compiler_params=params,
    )(xv, m1, s1col, t1col, m2)

    corr = None
    if npad != n:
        d = jnp.dot(m2, jnp.maximum(t1col, 0.0))[:, 0]
        extra = float(npad - n)
        corr = (extra * d, extra * d * d)
    s2col, t2col = _bn_cols(jnp.sum(s2p, axis=(0, 2)), jnp.sum(q2p, axis=(0, 2)),
                            float(n * p2), bn2_g, bn2_b, c2, p2, corr)

    logits_p, logp_p = pl.pallas_call(
        _p3_body,
        out_shape=(jax.ShapeDtypeStruct((npad, o), jnp.float32),
                   jax.ShapeDtypeStruct((npad, o), jnp.float32)),
        grid=(g,),
        in_specs=[pl.BlockSpec((f2, _JB, _SUB), lambda i: (0, i, 0)),
                  pl.BlockSpec((f2, 1), lambda i: (0, 0)),
                  pl.BlockSpec((f2, 1), lambda i: (0, 0)),
                  pl.BlockSpec((hdim, f2), lambda i: (0, 0)),
                  pl.BlockSpec((hdim, 1), lambda i: (0, 0)),
                  pl.BlockSpec((o, hdim), lambda i: (0, 0)),
                  pl.BlockSpec((o, 1), lambda i: (0, 0))],
        out_specs=(pl.BlockSpec((tile, o), lambda i: (i, 0)),
                   pl.BlockSpec((tile, o), lambda i: (i, 0))),
        compiler_params=params,
    )(y2, s2col, t2col, fc1_w, fc1_b.reshape(-1, 1),
      fc2_w, fc2_b.reshape(-1, 1))

    logits = logits_p[:n] if npad != n else logits_p
    logp = logp_p[:n] if npad != n else logp_p
    return {"output": logp, "logit": logits}


# speedup vs baseline: 3.3122x; 3.3122x over previous
import functools

import numpy as np
import jax
import jax.numpy as jnp
from jax import lax
from jax.experimental import pallas as pl
from jax.experimental.pallas import tpu as pltpu

_BN_EPS = 1e-5
_SUB = 128
_JB = 8
_VMEM_LIMIT = 64 * 1024 * 1024


@functools.lru_cache(None)
def _tap_selector(k, stride, pad, h, w):
    ho = (h + 2 * pad - k) // stride + 1
    wo = (w + 2 * pad - k) // stride + 1
    t = np.zeros((k * k, ho * wo, h * w), np.float32)
    for kh in range(k):
        for kw in range(k):
            for oy in range(ho):
                for ox in range(wo):
                    iy = oy * stride + kh - pad
                    ix = ox * stride + kw - pad
                    if 0 <= iy < h and 0 <= ix < w:
                        t[kh * k + kw, oy * wo + ox, iy * w + ix] = 1.0
    return t, ho, wo


@functools.lru_cache(None)
def _sel1_flat(k, stride, pad, h, w):
    t, ho, wo = _tap_selector(k, stride, pad, h, w)
    return t.reshape(k * k, -1), ho, wo


@functools.lru_cache(None)
def _sel2_flat(k, stride, pad, h, w, cin):
    t, ho, wo = _tap_selector(k, stride, pad, h, w)
    p, hw = ho * wo, h * w
    s = np.zeros((cin, k * k, p, cin, hw), np.float32)
    for d in range(cin):
        s[d, :, :, d, :] = t
    return s.reshape(cin * k * k, p * cin * hw), ho, wo


@functools.lru_cache(None)
def _group_mats(channels, positions):
    gs = np.zeros((channels, channels * positions), np.float32)
    for c in range(channels):
        gs[c, c * positions:(c + 1) * positions] = 1.0
    return gs, gs.T.copy()


def _round_up(a, m):
    return ((a + m - 1) // m) * m


def _make_body(g, tile, n, p1, p2, n_extra):
    inv1 = 1.0 / float(n * p1)
    inv2 = 1.0 / float(n * p2)

    def _body(x_ref, m1_ref, m2_ref, g1s_ref, g1b_ref, bg1_ref, bb1_ref,
              g2s_ref, g2b_ref, bg2_ref, bb2_ref, w1_ref, b1_ref,
              w2_ref, fb2_ref, lg_ref, lp_ref,
              z_st, y_st, s1_acc, q1_acc):
        i = pl.program_id(0)

        @pl.when(i == 0)
        def _init():
            s1_acc[...] = jnp.zeros_like(s1_acc)
            q1_acc[...] = jnp.zeros_like(q1_acc)

        xm = x_ref[...].reshape(x_ref.shape[0], -1).astype(jnp.bfloat16)
        z1 = jnp.dot(m1_ref[...], xm, preferred_element_type=jnp.float32)
        z_st[:, pl.ds(i * tile, tile)] = z1.astype(z_st.dtype)
        s1_acc[...] += jnp.sum(z1, axis=1, keepdims=True)
        q1_acc[...] += jnp.sum(z1 * z1, axis=1, keepdims=True)

        @pl.when(i == pl.num_programs(0) - 1)
        def _rest():
            s1c = jnp.dot(g1s_ref[...], s1_acc[...],
                          preferred_element_type=jnp.float32)
            q1c = jnp.dot(g1s_ref[...], q1_acc[...],
                          preferred_element_type=jnp.float32)
            mean1 = s1c * inv1
            var1 = q1c * inv1 - mean1 * mean1
            sc1 = bg1_ref[...] * lax.rsqrt(var1 + _BN_EPS)
            sh1 = bb1_ref[...] - mean1 * sc1
            s1col = jnp.dot(g1b_ref[...], sc1, preferred_element_type=jnp.float32)
            t1col = jnp.dot(g1b_ref[...], sh1, preferred_element_type=jnp.float32)

            s2 = jnp.zeros((y_st.shape[0], 1), jnp.float32)
            q2 = jnp.zeros((y_st.shape[0], 1), jnp.float32)
            for k in range(g):
                z1k = z_st[:, k * tile:(k + 1) * tile]
                a1 = jnp.maximum(z1k * s1col + t1col, 0.0)
                z2 = jnp.dot(m2_ref[...], a1.astype(jnp.bfloat16),
                             preferred_element_type=jnp.float32)
                y_st[:, k * tile:(k + 1) * tile] = z2.astype(y_st.dtype)
                s2 += jnp.sum(z2, axis=1, keepdims=True)
                q2 += jnp.sum(z2 * z2, axis=1, keepdims=True)

            if n_extra:
                d = jnp.dot(m2_ref[...],
                            jnp.maximum(t1col, 0.0).astype(jnp.bfloat16),
                            preferred_element_type=jnp.float32)
                s2 = s2 - n_extra * d
                q2 = q2 - n_extra * d * d

            s2c = jnp.dot(g2s_ref[...], s2, preferred_element_type=jnp.float32)
            q2c = jnp.dot(g2s_ref[...], q2, preferred_element_type=jnp.float32)
            mean2 = s2c * inv2
            var2 = q2c * inv2 - mean2 * mean2
            sc2 = bg2_ref[...] * lax.rsqrt(var2 + _BN_EPS)
            sh2 = bb2_ref[...] - mean2 * sc2
            s2col = jnp.dot(g2b_ref[...], sc2, preferred_element_type=jnp.float32)
            t2col = jnp.dot(g2b_ref[...], sh2, preferred_element_type=jnp.float32)

            for k in range(g):
                yk = y_st[:, k * tile:(k + 1) * tile]
                a2 = jnp.maximum(yk * s2col + t2col, 0.0)
                h = jnp.dot(w1_ref[...], a2,
                            preferred_element_type=jnp.float32) + b1_ref[...]
                lg = jnp.dot(w2_ref[...], h,
                             preferred_element_type=jnp.float32) + fb2_ref[...]
                m = jnp.max(lg, axis=0, keepdims=True)
                sh = lg - m
                lp = sh - jnp.log(jnp.sum(jnp.exp(sh), axis=0, keepdims=True))
                lg_ref[k * tile:(k + 1) * tile, :] = lg.T
                lp_ref[k * tile:(k + 1) * tile, :] = lp.T

    return _body


def kernel(x, conv1_w, conv1_b, bn1_g, bn1_b, conv2_w, conv2_b, bn2_g, bn2_b,
           fc1_w, fc1_b, fc2_w, fc2_b):
    n, cin, h, w = x.shape
    sel1, h1, w1 = _sel1_flat(3, 2, 1, h, w)
    c1 = conv1_w.shape[0]
    c2 = conv2_w.shape[0]
    sel2, h2, w2 = _sel2_flat(3, 2, 1, h1, w1, c1)
    p1 = h1 * w1
    p2 = h2 * w2
    f0 = cin * h * w
    f1 = c1 * p1
    f2 = c2 * p2
    o = fc2_w.shape[0]
    hdim = fc1_w.shape[0]
    tile = _JB * _SUB
    g1s, g1b = _group_mats(c1, p1)
    g2s, g2b = _group_mats(c2, p2)

    m1 = jnp.dot(conv1_w.reshape(c1, cin * 9).astype(jnp.bfloat16),
                 jnp.asarray(sel1, jnp.bfloat16)).reshape(f1, f0)
    m2 = jnp.dot(conv2_w.reshape(c2, c1 * 9).astype(jnp.bfloat16),
                 jnp.asarray(sel2, jnp.bfloat16)).reshape(f2, f1)

    npad = _round_up(n, tile)
    if npad == n:
        xv = jnp.transpose(x, (2, 3, 1, 0)).reshape(f0, n // _SUB, _SUB)
    else:
        xf = jnp.transpose(x, (2, 3, 1, 0)).reshape(f0, n)
        xv = jnp.pad(xf, ((0, 0), (0, npad - n))).reshape(f0, npad // _SUB, _SUB)
    g = npad // tile

    body = _make_body(g, tile, n, p1, p2, float(npad - n))
    const = lambda i: (0, 0)
    logits_p, logp_p = pl.pallas_call(
        body,
        out_shape=(jax.ShapeDtypeStruct((npad, o), jnp.float32),
                   jax.ShapeDtypeStruct((npad, o), jnp.float32)),
        grid=(g,),
        in_specs=[pl.BlockSpec((f0, _JB, _SUB), lambda i: (0, i, 0)),
                  pl.BlockSpec((f1, f0), const),
                  pl.BlockSpec((f2, f1), const),
                  pl.BlockSpec((c1, f1), const),
                  pl.BlockSpec((f1, c1), const),
                  pl.BlockSpec((c1, 1), const),
                  pl.BlockSpec((c1, 1), const),
                  pl.BlockSpec((c2, f2), const),
                  pl.BlockSpec((f2, c2), const),
                  pl.BlockSpec((c2, 1), const),
                  pl.BlockSpec((c2, 1), const),
                  pl.BlockSpec((hdim, f2), const),
                  pl.BlockSpec((hdim, 1), const),
                  pl.BlockSpec((o, hdim), const),
                  pl.BlockSpec((o, 1), const)],
        out_specs=(pl.BlockSpec((npad, o), const),
                   pl.BlockSpec((npad, o), const)),
        scratch_shapes=[pltpu.VMEM((f1, npad), jnp.bfloat16),
                        pltpu.VMEM((f2, npad), jnp.bfloat16),
                        pltpu.VMEM((f1, 1), jnp.float32),
                        pltpu.VMEM((f1, 1), jnp.float32)],
        compiler_params=pltpu.CompilerParams(
            dimension_semantics=("arbitrary",), vmem_limit_bytes=_VMEM_LIMIT),
    )(xv, m1, m2, jnp.asarray(g1s), jnp.asarray(g1b),
      bn1_g.reshape(-1, 1), bn1_b.reshape(-1, 1),
      jnp.asarray(g2s), jnp.asarray(g2b),
      bn2_g.reshape(-1, 1), bn2_b.reshape(-1, 1),
      fc1_w, fc1_b.reshape(-1, 1), fc2_w, fc2_b.reshape(-1, 1))

    logits = logits_p[:n] if npad != n else logits_p
    logp = logp_p[:n] if npad != n else logp_p
    return {"output": logp, "logit": logits}
